# baseline (speedup 1.0000x reference)
import jax
import jax.numpy as jnp
from jax.experimental import pallas as pl
from jax.experimental.pallas import tpu as pltpu
from jax.experimental.pallas import tpu_sc as plsc


def kernel(x, emb):
    seq_len = x.shape[1]
    hidden = emb.shape[1]
    mesh = plsc.VectorSubcoreMesh(core_axis_name="c", subcore_axis_name="s")
    n_workers = mesh.num_cores * mesh.num_subcores
    rows = seq_len // n_workers

    def body(emb_ref, out_ref):
        wid = jax.lax.axis_index("c") * mesh.num_subcores + jax.lax.axis_index("s")
        base = wid * rows
        pltpu.sync_copy(
            emb_ref.at[pl.ds(base, rows), :],
            out_ref.at[0, pl.ds(base, rows), :],
        )

    k = pl.kernel(
        body,
        out_type=jax.ShapeDtypeStruct((1, seq_len, hidden), emb.dtype),
        mesh=mesh,
    )
    return k(emb)


# speedup vs baseline: 1.0031x; 1.0031x over previous
import jax
import jax.numpy as jnp
from jax.experimental import pallas as pl
from jax.experimental.pallas import tpu as pltpu
from jax.experimental.pallas import tpu_sc as plsc

_CHUNKS = 8


def kernel(x, emb):
    seq_len = x.shape[1]
    hidden = emb.shape[1]
    mesh = plsc.VectorSubcoreMesh(core_axis_name="c", subcore_axis_name="s")
    n_workers = mesh.num_cores * mesh.num_subcores
    rows = seq_len // n_workers
    crows = rows // _CHUNKS

    def body(emb_ref, out_ref, sem):
        wid = jax.lax.axis_index("c") * mesh.num_subcores + jax.lax.axis_index("s")
        base = wid * rows
        copies = []
        for i in range(_CHUNKS):
            cp = pltpu.make_async_copy(
                emb_ref.at[pl.ds(base + i * crows, crows), :],
                out_ref.at[0, pl.ds(base + i * crows, crows), :],
                sem,
            )
            cp.start()
            copies.append(cp)
        for cp in copies:
            cp.wait()

    k = pl.kernel(
        body,
        out_type=jax.ShapeDtypeStruct((1, seq_len, hidden), emb.dtype),
        mesh=mesh,
        scratch_types=[pltpu.SemaphoreType.DMA],
    )
    return k(emb)


# speedup vs baseline: 80.6525x; 80.4040x over previous
import math

import jax
import jax.numpy as jnp
from jax.experimental import pallas as pl

_BLK = 512
_FINE = 64


def _sinusoid_block_kernel(fine_ref, out_ref):
    h = fine_ref.shape[1] // 2
    sub = _BLK // _FINE
    p0 = pl.program_id(0) * _BLK

    col = jax.lax.broadcasted_iota(jnp.int32, (sub, h), 1).astype(jnp.float32)
    row = jax.lax.broadcasted_iota(jnp.int32, (sub, h), 0).astype(jnp.float32)
    freq = jnp.exp((col * (1.0 / h)) * (-math.log(10000.0)))
    ang = (jnp.float32(p0) + row * jnp.float32(_FINE)) * freq
    cs = jnp.sin(ang)[:, None, :]
    cc = jnp.cos(ang)[:, None, :]

    fs = fine_ref[:, :h][None, :, :]
    fc = fine_ref[:, h:][None, :, :]

    out_ref[0, :, :] = jnp.concatenate(
        [
            (fs * cc + fc * cs).reshape(_BLK, h),
            (fc * cc - fs * cs).reshape(_BLK, h),
        ],
        axis=1,
    )


def kernel(x, emb):
    seq_len = x.shape[1]
    hidden = emb.shape[1]
    grid = seq_len // _BLK
    return pl.pallas_call(
        _sinusoid_block_kernel,
        grid=(grid,),
        in_specs=[
            pl.BlockSpec((_FINE, hidden), lambda i: (0, 0)),
        ],
        out_specs=pl.BlockSpec((1, _BLK, hidden), lambda i: (0, i, 0)),
        out_shape=jax.ShapeDtypeStruct((1, seq_len, hidden), emb.dtype),
    )(emb)
